# baseline (speedup 1.0000x reference)
import jax
import jax.numpy as jnp
from jax.experimental import pallas as pl
from jax.experimental.pallas import tpu as pltpu

B, C, H, W = 16, 384, 64, 64
D = H * W
D0, D1 = 8, D // 8
R = B * C
RB = 8


def _copy_body(idx_ref, *refs):
    ins = refs[:RB]
    out = refs[RB]
    for j in range(RB):
        out[pl.ds(j, 1)] = ins[j][...]


@jax.jit
def _shuffle(x3d, row_idx):
    grid_spec = pltpu.PrefetchScalarGridSpec(
        num_scalar_prefetch=1,
        grid=(R // RB,),
        in_specs=[
            pl.BlockSpec((1, D0, D1), lambda r, idx, j=j: (idx[RB * r + j], 0, 0))
            for j in range(RB)
        ],
        out_specs=pl.BlockSpec((RB, D0, D1), lambda r, idx: (r, 0, 0)),
    )
    f = pl.pallas_call(
        _copy_body,
        grid_spec=grid_spec,
        out_shape=jax.ShapeDtypeStruct((R, D0, D1), jnp.float32),
        compiler_params=pltpu.CompilerParams(
            dimension_semantics=("arbitrary",),
        ),
    )
    return f(row_idx, *([x3d] * RB))


def kernel(inputs, permutation):
    x3d = inputs.reshape(R, D0, D1)
    perm32 = permutation.astype(jnp.int32)
    row_idx = (perm32[None, :] + C * jnp.arange(B, dtype=jnp.int32)[:, None]).reshape(-1)
    return _shuffle(x3d, row_idx).reshape(B, C, H, W)
